# Initial kernel scaffold; baseline (speedup 1.0000x reference)
import jax
import jax.numpy as jnp
from jax.experimental import pallas as pl


def kernel(x_emb, atomic_numbers, edge_distance, edge_index, wigner, wigner_inv, to_grid, from_grid, source_embedding, target_embedding, rad1_w1, rad1_b1, rad1_ln1_g, rad1_ln1_b, rad1_w2, rad1_b2, rad1_ln2_g, rad1_ln2_b, rad1_w3, rad1_b3, conv1_w0, conv1_b0, conv1_w1, conv1_w2, conv2_w0, conv2_b0, conv2_w1, conv2_w2, alpha_ln_g, alpha_ln_b, alpha_dot, proj_w, proj_b):
    raise NotImplementedError("write your pallas kernel here")


# speedup vs baseline: 2.8066x; 2.8066x over previous
import math
import numpy as np

import jax
import jax.numpy as jnp
from jax.experimental import pallas as pl
from jax.experimental.pallas import tpu as pltpu

LMAX = 2
MMAX = 2
K = (LMAX + 1) ** 2
SPHERE_CH = 8
HIDDEN_CH = 8
NUM_HEADS = 2
ATTN_ALPHA_CH = 4
ATTN_VALUE_CH = 4
OUTPUT_CH = 8
SILU_SCALE = 1.0 / 0.6

C_IN1 = 2 * SPHERE_CH
ALPHA_TOT = NUM_HEADS * ATTN_ALPHA_CH
VALUE_TOT = NUM_HEADS * ATTN_VALUE_CH
EXTRA_M0 = ALPHA_TOT + HIDDEN_CH

EDGE_TILE = 512

MSG_W = K * VALUE_TOT
PACK_W = 128
PAD_W = PACK_W - MSG_W - NUM_HEADS

PROJ_W = K * OUTPUT_CH
PROJ_PACK_W = 128

M_IDX = [([l * l + l for l in range(LMAX + 1)], [])]
for _m in range(1, MMAX + 1):
    M_IDX.append(([l * l + l + _m for l in range(_m, LMAX + 1)],
                  [l * l + l - _m for l in range(_m, LMAX + 1)]))

L_PER_COEF = np.concatenate([[l] * (2 * l + 1) for l in range(LMAX + 1)]).astype(np.int32)


def _scaled_silu(x):
    return x * jax.nn.sigmoid(x) * SILU_SCALE


def _layer_norm(x, g, b, eps=1e-5):
    mu = jnp.mean(x, axis=-1, keepdims=True)
    var = jnp.mean((x - mu) ** 2, axis=-1, keepdims=True)
    return (x - mu) * jax.lax.rsqrt(var + eps) * g + b


def _smooth_leaky_relu(x, alpha=0.2):
    return ((1.0 + alpha) / 2.0) * x + ((1.0 - alpha) / 2.0) * x * (2.0 * jax.nn.sigmoid(x) - 1.0)


def _rotate_coefs(wig_flat, coefs):
    out = []
    for k in range(K):
        acc = wig_flat[:, k * K:k * K + 1] * coefs[0]
        for j in range(1, K):
            acc = acc + wig_flat[:, k * K + j:k * K + j + 1] * coefs[j]
        out.append(acc)
    return out


def _so2_conv_coefs(coefs, w_list, b0, c_in, m_out, rad=None, extra=0):
    out = [None] * K
    f32 = jnp.float32
    idx0 = M_IDX[0][0]
    x0 = jnp.concatenate([coefs[i] for i in idx0], axis=-1)
    off = len(idx0) * c_in
    if rad is not None:
        x0 = x0 * rad[:, :off]
    y0 = jnp.dot(x0, w_list[0], preferred_element_type=f32) + b0
    x_extra = None
    if extra:
        x_extra = y0[:, :extra]
        y0 = y0[:, extra:]
    for t, i in enumerate(idx0):
        out[i] = y0[:, t * m_out:(t + 1) * m_out]
    for m in range(1, MMAX + 1):
        plus_idx, minus_idx = M_IDX[m]
        nm = len(plus_idx)
        in_w = nm * c_in
        half = nm * m_out
        xp = jnp.concatenate([coefs[i] for i in plus_idx], axis=-1)
        xm = jnp.concatenate([coefs[i] for i in minus_idx], axis=-1)
        if rad is not None:
            r = rad[:, off:off + in_w]
            xp = xp * r
            xm = xm * r
        off += in_w
        yp = jnp.dot(xp, w_list[m], preferred_element_type=f32)
        ym = jnp.dot(xm, w_list[m], preferred_element_type=f32)
        op = yp[:, :half] - ym[:, half:]
        om = yp[:, half:] + ym[:, :half]
        for t, i in enumerate(plus_idx):
            out[i] = op[:, t * m_out:(t + 1) * m_out]
        for t, i in enumerate(minus_idx):
            out[i] = om[:, t * m_out:(t + 1) * m_out]
    return out, x_extra


def _s2_act_coefs(coefs, gating, tg_exp, fg_exp, ch):
    x = jnp.concatenate(coefs, axis=-1)
    g = _scaled_silu(jnp.dot(x, tg_exp, preferred_element_type=jnp.float32))
    y = jnp.dot(g, fg_exp, preferred_element_type=jnp.float32)
    out = [_scaled_silu(gating)]
    for k in range(1, K):
        out.append(y[:, k * ch:(k + 1) * ch])
    return out


def _fused_edge_kernel(
        x_edge_ref, x_msg_ref, wig_ref, wiginv_ref,
        rw1_ref, rb1_ref, rg1_ref, rbe1_ref,
        rw2_ref, rb2_ref, rg2_ref, rbe2_ref,
        rw3_ref, rb3_ref,
        c1w0_ref, c1b0_ref, c1w1_ref, c1w2_ref,
        tgx_ref, fgx_ref,
        c2w0_ref, c2b0_ref, c2w1_ref, c2w2_ref,
        ag_ref, ab_ref, adot_ref,
        out_ref):
    f32 = jnp.float32
    te = x_edge_ref.shape[0]

    h = jnp.dot(x_edge_ref[...], rw1_ref[...], preferred_element_type=f32) + rb1_ref[...]
    h = _scaled_silu(_layer_norm(h, rg1_ref[...], rbe1_ref[...]))
    h = jnp.dot(h, rw2_ref[...], preferred_element_type=f32) + rb2_ref[...]
    h = _scaled_silu(_layer_norm(h, rg2_ref[...], rbe2_ref[...]))
    rad = jnp.dot(h, rw3_ref[...], preferred_element_type=f32) + rb3_ref[...]

    xin = x_msg_ref[...]
    in_coefs = [xin[:, j * C_IN1:(j + 1) * C_IN1] for j in range(K)]
    rot = _rotate_coefs(wig_ref[...], in_coefs)

    hid, extra = _so2_conv_coefs(
        rot, [c1w0_ref[...], c1w1_ref[...], c1w2_ref[...]], c1b0_ref[...],
        c_in=C_IN1, m_out=HIDDEN_CH, rad=rad, extra=EXTRA_M0)
    alpha_feat = extra[:, :ALPHA_TOT]
    gating = extra[:, ALPHA_TOT:]

    act = _s2_act_coefs(hid, gating, tgx_ref[...], fgx_ref[...], HIDDEN_CH)

    val, _ = _so2_conv_coefs(
        act, [c2w0_ref[...], c2w1_ref[...], c2w2_ref[...]], c2b0_ref[...],
        c_in=HIDDEN_CH, m_out=VALUE_TOT, rad=None, extra=0)

    adot = adot_ref[...]
    cols = []
    for hd in range(NUM_HEADS):
        a = alpha_feat[:, hd * ATTN_ALPHA_CH:(hd + 1) * ATTN_ALPHA_CH]
        a = _smooth_leaky_relu(_layer_norm(a, ag_ref[...], ab_ref[...]))
        cols.append(jnp.sum(a * adot[hd:hd + 1, :], axis=-1, keepdims=True))
    alpha = jnp.concatenate(cols, axis=-1)

    out_coefs = _rotate_coefs(wiginv_ref[...], val)
    msg = jnp.concatenate(out_coefs, axis=-1)

    out_ref[...] = jnp.concatenate(
        [msg, alpha, jnp.zeros((te, PAD_W), f32)], axis=-1)


def _fused_edge_messages(x_edge, x_msg, wig2, wiginv2, weights, te=EDGE_TILE):
    E = x_edge.shape[0]
    grid = (E // te,)

    def row_spec(width):
        return pl.BlockSpec((te, width), lambda i: (i, 0))

    in_specs = [row_spec(x_edge.shape[1]), row_spec(x_msg.shape[1]),
                row_spec(wig2.shape[1]), row_spec(wiginv2.shape[1])]
    in_specs += [pl.BlockSpec(w.shape, lambda i: (0, 0)) for w in weights]

    out_shape = jax.ShapeDtypeStruct((E, PACK_W), jnp.float32)
    out_specs = pl.BlockSpec((te, PACK_W), lambda i: (i, 0))

    packed = pl.pallas_call(
        _fused_edge_kernel,
        out_shape=out_shape,
        grid=grid,
        in_specs=in_specs,
        out_specs=out_specs,
        compiler_params=pltpu.CompilerParams(
            dimension_semantics=("parallel",),
            vmem_limit_bytes=64 * 1024 * 1024),
    )(x_edge, x_msg, wig2, wiginv2, *weights)

    msg = packed[:, :MSG_W]
    alpha = packed[:, MSG_W:MSG_W + NUM_HEADS]
    return msg, alpha


def _node_proj_kernel(acc_ref, w_ref, b_ref, o_ref):
    acc = acc_ref[...]
    x = acc[:, :MSG_W]
    z = acc[:, MSG_W:MSG_W + NUM_HEADS]
    inv = 1.0 / (z + 1e-16)
    parts = []
    for hd in range(NUM_HEADS):
        col = inv[:, hd:hd + 1]
        parts.append(jnp.broadcast_to(col, (col.shape[0], ATTN_VALUE_CH)))
    block = jnp.concatenate(parts, axis=1)
    inv_full = jnp.concatenate([block] * K, axis=1)
    o_ref[...] = jnp.dot(x * inv_full, w_ref[...],
                         preferred_element_type=jnp.float32) + b_ref[...]


def _node_divide_project(acc, wbd_pad, bias_row):
    N = acc.shape[0]
    return pl.pallas_call(
        _node_proj_kernel,
        out_shape=jax.ShapeDtypeStruct((N, PROJ_PACK_W), jnp.float32),
        grid=(1,),
        in_specs=[pl.BlockSpec((N, PACK_W), lambda i: (0, 0)),
                  pl.BlockSpec((MSG_W, PROJ_PACK_W), lambda i: (0, 0)),
                  pl.BlockSpec((1, PROJ_PACK_W), lambda i: (0, 0))],
        out_specs=pl.BlockSpec((N, PROJ_PACK_W), lambda i: (0, 0)),
    )(acc, wbd_pad, bias_row)


def kernel(x_emb, atomic_numbers, edge_distance, edge_index, wigner, wigner_inv,
           to_grid, from_grid, source_embedding, target_embedding,
           rad1_w1, rad1_b1, rad1_ln1_g, rad1_ln1_b, rad1_w2, rad1_b2,
           rad1_ln2_g, rad1_ln2_b, rad1_w3, rad1_b3,
           conv1_w0, conv1_b0, conv1_w1, conv1_w2,
           conv2_w0, conv2_b0, conv2_w1, conv2_w2,
           alpha_ln_g, alpha_ln_b, alpha_dot, proj_w, proj_b):
    E = edge_index.shape[1]
    N = x_emb.shape[0]
    src, tgt = edge_index[0], edge_index[1]

    src_e = source_embedding[atomic_numbers[src]]
    tgt_e = target_embedding[atomic_numbers[tgt]]
    x_edge = jnp.concatenate([edge_distance, src_e, tgt_e], axis=1)

    x_msg = jnp.concatenate([x_emb[src], x_emb[tgt]], axis=2).reshape(E, K * C_IN1)
    wig2 = wigner.reshape(E, K * K)
    wiginv2 = wigner_inv.reshape(E, K * K)

    eye_h = jnp.eye(HIDDEN_CH, dtype=jnp.float32)
    tg_exp = jnp.kron(to_grid.T, eye_h)
    fg_exp = jnp.kron(from_grid, eye_h)

    _r = lambda v: v.reshape(1, -1)
    weights = [
        rad1_w1, _r(rad1_b1), _r(rad1_ln1_g), _r(rad1_ln1_b),
        rad1_w2, _r(rad1_b2), _r(rad1_ln2_g), _r(rad1_ln2_b),
        rad1_w3, _r(rad1_b3),
        conv1_w0, _r(conv1_b0), conv1_w1, conv1_w2,
        tg_exp, fg_exp,
        conv2_w0, _r(conv2_b0), conv2_w1, conv2_w2,
        _r(alpha_ln_g), _r(alpha_ln_b), alpha_dot,
    ]

    msg, alpha = _fused_edge_messages(x_edge, x_msg, wig2, wiginv2, weights)

    seg_max = jax.ops.segment_max(alpha, tgt, num_segments=N)
    w = jnp.exp(alpha - seg_max[tgt])
    weighted = (msg.reshape(E, K, NUM_HEADS, ATTN_VALUE_CH)
                * w[:, None, :, None]).reshape(E, MSG_W)
    packed = jnp.concatenate(
        [weighted, w, jnp.zeros((E, PAD_W), jnp.float32)], axis=1)
    acc = jax.ops.segment_sum(packed, tgt, num_segments=N)

    w_per = jnp.transpose(proj_w[L_PER_COEF], (0, 2, 1))
    eye_k = jnp.eye(K, dtype=jnp.float32)
    wbd = (eye_k[:, None, :, None] * w_per[:, :, None, :]).reshape(MSG_W, PROJ_W)
    wbd_pad = jnp.zeros((MSG_W, PROJ_PACK_W), jnp.float32).at[:, :PROJ_W].set(wbd)
    bias_row = jnp.zeros((1, PROJ_PACK_W), jnp.float32).at[0, :OUTPUT_CH].set(proj_b)
    out = _node_divide_project(acc, wbd_pad, bias_row)[:, :PROJ_W]
    return out.reshape(N, K, OUTPUT_CH)


# speedup vs baseline: 11.7494x; 4.1863x over previous
import math
import numpy as np

import jax
import jax.numpy as jnp
from jax.experimental import pallas as pl
from jax.experimental.pallas import tpu as pltpu

LMAX = 2
MMAX = 2
K = (LMAX + 1) ** 2
SPHERE_CH = 8
HIDDEN_CH = 8
NUM_HEADS = 2
ATTN_ALPHA_CH = 4
ATTN_VALUE_CH = 4
OUTPUT_CH = 8
SILU_SCALE = 1.0 / 0.6

C_IN1 = 2 * SPHERE_CH
ALPHA_TOT = NUM_HEADS * ATTN_ALPHA_CH
VALUE_TOT = NUM_HEADS * ATTN_VALUE_CH
EXTRA_M0 = ALPHA_TOT + HIDDEN_CH

EDGE_TILE = 512

MSG_W = K * VALUE_TOT
PACK_W = 128
PAD_W = PACK_W - MSG_W - NUM_HEADS

PROJ_W = K * OUTPUT_CH
PROJ_PACK_W = 128

NODE_W = 128
X_W = K * C_IN1

M_IDX = [([l * l + l for l in range(LMAX + 1)], [])]
for _m in range(1, MMAX + 1):
    M_IDX.append(([l * l + l + _m for l in range(_m, LMAX + 1)],
                  [l * l + l - _m for l in range(_m, LMAX + 1)]))

L_PER_COEF = np.concatenate([[l] * (2 * l + 1) for l in range(LMAX + 1)]).astype(np.int32)


def _expansion_mats():
    prot = np.zeros((K, K * K, X_W), np.float32)
    trot = np.zeros((K, X_W, X_W), np.float32)
    pinv = np.zeros((K, K * K, MSG_W), np.float32)
    tinv = np.zeros((K, MSG_W, MSG_W), np.float32)
    for j in range(K):
        for k in range(K):
            prot[j, j * K + k, k * C_IN1:(k + 1) * C_IN1] = 1.0
            pinv[j, j * K + k, k * VALUE_TOT:(k + 1) * VALUE_TOT] = 1.0
            for c in range(SPHERE_CH):
                trot[j, j * SPHERE_CH + c, k * C_IN1 + c] = 1.0
                trot[j, K * SPHERE_CH + j * SPHERE_CH + c,
                     k * C_IN1 + SPHERE_CH + c] = 1.0
            for c in range(VALUE_TOT):
                tinv[j, j * VALUE_TOT + c, k * VALUE_TOT + c] = 1.0
    return prot, trot, pinv, tinv


_PROT, _TROT, _PINV, _TINV = _expansion_mats()


def _scaled_silu(x):
    return x * jax.nn.sigmoid(x) * SILU_SCALE


def _layer_norm(x, g, b, eps=1e-5):
    mu = jnp.mean(x, axis=-1, keepdims=True)
    var = jnp.mean((x - mu) ** 2, axis=-1, keepdims=True)
    return (x - mu) * jax.lax.rsqrt(var + eps) * g + b


def _smooth_leaky_relu(x, alpha=0.2):
    return ((1.0 + alpha) / 2.0) * x + ((1.0 - alpha) / 2.0) * x * (2.0 * jax.nn.sigmoid(x) - 1.0)


def _so2_conv_coefs(coefs, w_list, b0, c_in, m_out, rad=None, extra=0):
    out = [None] * K
    f32 = jnp.float32
    idx0 = M_IDX[0][0]
    x0 = jnp.concatenate([coefs[i] for i in idx0], axis=-1)
    off = len(idx0) * c_in
    if rad is not None:
        x0 = x0 * rad[:, :off]
    y0 = jnp.dot(x0, w_list[0], preferred_element_type=f32) + b0
    x_extra = None
    if extra:
        x_extra = y0[:, :extra]
        y0 = y0[:, extra:]
    for t, i in enumerate(idx0):
        out[i] = y0[:, t * m_out:(t + 1) * m_out]
    for m in range(1, MMAX + 1):
        plus_idx, minus_idx = M_IDX[m]
        nm = len(plus_idx)
        in_w = nm * c_in
        half = nm * m_out
        xp = jnp.concatenate([coefs[i] for i in plus_idx], axis=-1)
        xm = jnp.concatenate([coefs[i] for i in minus_idx], axis=-1)
        if rad is not None:
            r = rad[:, off:off + in_w]
            xp = xp * r
            xm = xm * r
        off += in_w
        yp = jnp.dot(xp, w_list[m], preferred_element_type=f32)
        ym = jnp.dot(xm, w_list[m], preferred_element_type=f32)
        op = yp[:, :half] - ym[:, half:]
        om = yp[:, half:] + ym[:, :half]
        for t, i in enumerate(plus_idx):
            out[i] = op[:, t * m_out:(t + 1) * m_out]
        for t, i in enumerate(minus_idx):
            out[i] = om[:, t * m_out:(t + 1) * m_out]
    return out, x_extra


def _s2_act_coefs(coefs, gating, tg_exp, fg_exp, ch):
    x = jnp.concatenate(coefs, axis=-1)
    g = _scaled_silu(jnp.dot(x, tg_exp, preferred_element_type=jnp.float32))
    y = jnp.dot(g, fg_exp, preferred_element_type=jnp.float32)
    out = [_scaled_silu(gating)]
    for k in range(1, K):
        out.append(y[:, k * ch:(k + 1) * ch])
    return out


def _fused_edge_kernel(
        idx_ref, ed_ref, wig_ref, wiginv_ref, ntab_ref,
        rw1_ref, rb1_ref, rg1_ref, rbe1_ref,
        rw2_ref, rb2_ref, rg2_ref, rbe2_ref,
        rw3_ref, rb3_ref,
        c1w0_ref, c1b0_ref, c1w1_ref, c1w2_ref,
        tgx_ref, fgx_ref,
        c2w0_ref, c2b0_ref, c2w1_ref, c2w2_ref,
        ag_ref, ab_ref, adot_ref,
        prot_ref, trot_ref, pinv_ref, tinv_ref,
        out_ref, src_rows, tgt_rows):
    f32 = jnp.float32
    te = EDGE_TILE

    for mi in range(te):
        src_rows[mi] = ntab_ref[idx_ref[0, 0, mi], 0]
        tgt_rows[mi] = ntab_ref[idx_ref[0, 1, mi], 0]
    xs = src_rows[...]
    xt = tgt_rows[...]

    x_edge = jnp.concatenate(
        [ed_ref[...], xs[:, 72:88], xt[:, 88:104]], axis=1)
    h = jnp.dot(x_edge, rw1_ref[...], preferred_element_type=f32) + rb1_ref[...]
    h = _scaled_silu(_layer_norm(h, rg1_ref[...], rbe1_ref[...]))
    h = jnp.dot(h, rw2_ref[...], preferred_element_type=f32) + rb2_ref[...]
    h = _scaled_silu(_layer_norm(h, rg2_ref[...], rbe2_ref[...]))
    rad = jnp.dot(h, rw3_ref[...], preferred_element_type=f32) + rb3_ref[...]

    X = jnp.concatenate([xs[:, :72], xt[:, :72]], axis=1)
    wigi = wiginv_ref[...]
    rot_cat = None
    for j in range(K):
        d = jnp.dot(wigi, prot_ref[j], preferred_element_type=f32)
        v = jnp.dot(X, trot_ref[j], preferred_element_type=f32)
        rot_cat = d * v if rot_cat is None else rot_cat + d * v
    rot = [rot_cat[:, i * C_IN1:(i + 1) * C_IN1] for i in range(K)]

    hid, extra = _so2_conv_coefs(
        rot, [c1w0_ref[...], c1w1_ref[...], c1w2_ref[...]], c1b0_ref[...],
        c_in=C_IN1, m_out=HIDDEN_CH, rad=rad, extra=EXTRA_M0)
    alpha_feat = extra[:, :ALPHA_TOT]
    gating = extra[:, ALPHA_TOT:]

    act = _s2_act_coefs(hid, gating, tgx_ref[...], fgx_ref[...], HIDDEN_CH)

    val, _ = _so2_conv_coefs(
        act, [c2w0_ref[...], c2w1_ref[...], c2w2_ref[...]], c2b0_ref[...],
        c_in=HIDDEN_CH, m_out=VALUE_TOT, rad=None, extra=0)

    adot = adot_ref[...]
    cols = []
    for hd in range(NUM_HEADS):
        a = alpha_feat[:, hd * ATTN_ALPHA_CH:(hd + 1) * ATTN_ALPHA_CH]
        a = _smooth_leaky_relu(_layer_norm(a, ag_ref[...], ab_ref[...]))
        cols.append(jnp.sum(a * adot[hd:hd + 1, :], axis=-1, keepdims=True))
    alpha = jnp.concatenate(cols, axis=-1)

    vcat = jnp.concatenate(val, axis=1)
    wigf = wig_ref[...]
    msg = None
    for j in range(K):
        d = jnp.dot(wigf, pinv_ref[j], preferred_element_type=f32)
        v = jnp.dot(vcat, tinv_ref[j], preferred_element_type=f32)
        msg = d * v if msg is None else msg + d * v

    w = jnp.exp(alpha)
    parts = []
    for hd in range(NUM_HEADS):
        col = w[:, hd:hd + 1]
        parts.append(jnp.broadcast_to(col, (te, ATTN_VALUE_CH)))
    block = jnp.concatenate(parts, axis=1)
    wfull = jnp.concatenate([block] * K, axis=1)

    out_ref[...] = jnp.concatenate(
        [msg * wfull, w, jnp.zeros((te, PAD_W), f32)], axis=-1)


def _fused_edge_messages(idx2, edge_distance, wig2, wiginv2, node_tab, weights):
    E = wig2.shape[0]
    te = EDGE_TILE
    grid = (E // te,)

    def row_spec(width):
        return pl.BlockSpec((te, width), lambda i: (i, 0))

    in_specs = [
        pl.BlockSpec((1, 2, te), lambda i: (i, 0, 0), memory_space=pltpu.SMEM),
        row_spec(edge_distance.shape[1]),
        row_spec(wig2.shape[1]), row_spec(wiginv2.shape[1]),
        pl.BlockSpec(node_tab.shape, lambda i: (0, 0, 0)),
    ]
    in_specs += [pl.BlockSpec(w.shape, lambda i, n=w.ndim: (0,) * n)
                 for w in weights]

    out_shape = jax.ShapeDtypeStruct((E, PACK_W), jnp.float32)
    out_specs = pl.BlockSpec((te, PACK_W), lambda i: (i, 0))

    return pl.pallas_call(
        _fused_edge_kernel,
        out_shape=out_shape,
        grid=grid,
        in_specs=in_specs,
        out_specs=out_specs,
        scratch_shapes=[pltpu.VMEM((te, NODE_W), jnp.float32),
                        pltpu.VMEM((te, NODE_W), jnp.float32)],
        compiler_params=pltpu.CompilerParams(
            dimension_semantics=("parallel",),
            vmem_limit_bytes=96 * 1024 * 1024),
    )(idx2, edge_distance, wig2, wiginv2, node_tab, *weights)


def _node_proj_kernel(acc_ref, w_ref, b_ref, o_ref):
    acc = acc_ref[...]
    x = acc[:, :MSG_W]
    z = acc[:, MSG_W:MSG_W + NUM_HEADS]
    inv = 1.0 / (z + 1e-16)
    parts = []
    for hd in range(NUM_HEADS):
        col = inv[:, hd:hd + 1]
        parts.append(jnp.broadcast_to(col, (col.shape[0], ATTN_VALUE_CH)))
    block = jnp.concatenate(parts, axis=1)
    inv_full = jnp.concatenate([block] * K, axis=1)
    o_ref[...] = jnp.dot(x * inv_full, w_ref[...],
                         preferred_element_type=jnp.float32) + b_ref[...]


def _node_divide_project(acc, wbd_pad, bias_row):
    N = acc.shape[0]
    return pl.pallas_call(
        _node_proj_kernel,
        out_shape=jax.ShapeDtypeStruct((N, PROJ_PACK_W), jnp.float32),
        grid=(1,),
        in_specs=[pl.BlockSpec((N, PACK_W), lambda i: (0, 0)),
                  pl.BlockSpec((MSG_W, PROJ_PACK_W), lambda i: (0, 0)),
                  pl.BlockSpec((1, PROJ_PACK_W), lambda i: (0, 0))],
        out_specs=pl.BlockSpec((N, PROJ_PACK_W), lambda i: (0, 0)),
    )(acc, wbd_pad, bias_row)


def kernel(x_emb, atomic_numbers, edge_distance, edge_index, wigner, wigner_inv,
           to_grid, from_grid, source_embedding, target_embedding,
           rad1_w1, rad1_b1, rad1_ln1_g, rad1_ln1_b, rad1_w2, rad1_b2,
           rad1_ln2_g, rad1_ln2_b, rad1_w3, rad1_b3,
           conv1_w0, conv1_b0, conv1_w1, conv1_w2,
           conv2_w0, conv2_b0, conv2_w1, conv2_w2,
           alpha_ln_g, alpha_ln_b, alpha_dot, proj_w, proj_b):
    E = edge_index.shape[1]
    N = x_emb.shape[0]
    te = EDGE_TILE
    src, tgt = edge_index[0], edge_index[1]

    node_tab = jnp.concatenate(
        [x_emb.reshape(N, K * SPHERE_CH),
         source_embedding[atomic_numbers],
         target_embedding[atomic_numbers],
         jnp.zeros((N, NODE_W - K * SPHERE_CH - 32), jnp.float32)],
        axis=1).reshape(N, 1, NODE_W)

    idx2 = jnp.stack([src, tgt], axis=0).reshape(2, E // te, te).transpose(1, 0, 2)

    wig2 = wigner.reshape(E, K * K)
    wiginv2 = wigner_inv.reshape(E, K * K)

    eye_h = jnp.eye(HIDDEN_CH, dtype=jnp.float32)
    tg_exp = jnp.kron(to_grid.T, eye_h)
    fg_exp = jnp.kron(from_grid, eye_h)

    _r = lambda v: v.reshape(1, -1)
    weights = [
        rad1_w1, _r(rad1_b1), _r(rad1_ln1_g), _r(rad1_ln1_b),
        rad1_w2, _r(rad1_b2), _r(rad1_ln2_g), _r(rad1_ln2_b),
        rad1_w3, _r(rad1_b3),
        conv1_w0, _r(conv1_b0), conv1_w1, conv1_w2,
        tg_exp, fg_exp,
        conv2_w0, _r(conv2_b0), conv2_w1, conv2_w2,
        _r(alpha_ln_g), _r(alpha_ln_b), alpha_dot,
        jnp.asarray(_PROT), jnp.asarray(_TROT),
        jnp.asarray(_PINV), jnp.asarray(_TINV),
    ]

    packed = _fused_edge_messages(idx2, edge_distance, wig2, wiginv2,
                                  node_tab, weights)

    acc = jax.ops.segment_sum(packed, tgt, num_segments=N)

    w_per = jnp.transpose(proj_w[L_PER_COEF], (0, 2, 1))
    eye_k = jnp.eye(K, dtype=jnp.float32)
    wbd = (eye_k[:, None, :, None] * w_per[:, :, None, :]).reshape(MSG_W, PROJ_W)
    wbd_pad = jnp.zeros((MSG_W, PROJ_PACK_W), jnp.float32).at[:, :PROJ_W].set(wbd)
    bias_row = jnp.zeros((1, PROJ_PACK_W), jnp.float32).at[0, :OUTPUT_CH].set(proj_b)
    out = _node_divide_project(acc, wbd_pad, bias_row)[:, :PROJ_W]
    return out.reshape(N, K, OUTPUT_CH)
